# baseline (speedup 1.0000x reference)
import functools

import jax
import jax.numpy as jnp
from jax import lax
from jax.experimental import pallas as pl
from jax.experimental.pallas import tpu as pltpu
from jax.experimental.pallas import tpu_sc as plsc

VOCAB = 1000
D_EMB = 128
BATCH = 4096
HIST = 200

NC, NS = 2, 16
NW = NC * NS
ROWS = BATCH * HIST
RPW = ROWS // NW
CHUNK = 128
NCHUNK = RPW // CHUNK
NBUF = 4


def _sc_body(table_hbm, idx_hbm, out_hbm, table_sp, idx_v,
             buf0, buf1, buf2, buf3, g0, g1, g2, g3, o0, o1, o2, o3):
    sid = lax.axis_index("s")
    wid = sid * NC + lax.axis_index("c")
    bufs = (buf0, buf1, buf2, buf3)
    gsems = (g0, g1, g2, g3)
    osems = (o0, o1, o2, o3)
    cbase = wid * NCHUNK

    @pl.when(sid == 0)
    def _():
        pltpu.sync_copy(table_hbm, table_sp)

    pltpu.sync_copy(idx_hbm.at[wid], idx_v)
    plsc.subcore_barrier()

    def g_src(b):
        return table_hbm if b % NBUF == 3 else table_sp

    def g_start(g, b):
        pltpu.async_copy(g_src(b).at[idx_v.at[g]], bufs[b], gsems[b])

    def g_wait(g, b):
        pltpu.make_async_copy(g_src(b).at[idx_v.at[g]], bufs[b], gsems[b]).wait()

    def o_start(g, b):
        pltpu.async_copy(bufs[b], out_hbm.at[cbase + g], osems[b])

    def o_wait(g, b):
        pltpu.make_async_copy(bufs[b], out_hbm.at[cbase + g], osems[b]).wait()

    g_start(0, 0)
    g_start(1, 1)

    for g in range(NBUF):
        g_wait(g, g % NBUF)
        o_start(g, g % NBUF)
        if g >= 2:
            o_wait(g - 2, (g + 2) % NBUF)
        g_start(g + 2, (g + 2) % NBUF)

    def loop_body(i, carry):
        for b in range(NBUF):
            g = i * NBUF + b
            b2 = (b + 2) % NBUF
            g_wait(g, b)
            o_start(g, b)
            o_wait(g - 2, b2)
            g_start(g + 2, b2)
        return carry

    lax.fori_loop(1, NCHUNK // NBUF - 1, loop_body, None)

    for b in range(NBUF):
        g = NCHUNK - NBUF + b
        g_wait(g, b)
        o_start(g, b)
        if g + 2 < NCHUNK:
            o_wait(g - 2, (g + 2) % NBUF)
            g_start(g + 2, (g + 2) % NBUF)

    for b in range(NBUF):
        g = NCHUNK - NBUF + b
        o_wait(g, b)


_sc_gather = functools.partial(
    pl.kernel,
    out_type=jax.ShapeDtypeStruct((NW * NCHUNK, CHUNK, D_EMB), jnp.float32),
    mesh=plsc.VectorSubcoreMesh(core_axis_name="c", subcore_axis_name="s"),
    scratch_types=(
        [pltpu.VMEM_SHARED((VOCAB, D_EMB), jnp.float32)]
        + [pltpu.VMEM((NCHUNK, CHUNK), jnp.int32)]
        + [pltpu.VMEM((CHUNK, D_EMB), jnp.float32)] * NBUF
        + [pltpu.SemaphoreType.DMA] * (2 * NBUF)
    ),
)(_sc_body)


def kernel(x, table):
    idx = x.astype(jnp.int32).reshape(NW, NCHUNK, CHUNK)
    out = _sc_gather(table, idx)
    return out.reshape(BATCH, HIST, D_EMB)


# speedup vs baseline: 1.2318x; 1.2318x over previous
import functools

import jax
import jax.numpy as jnp
from jax import lax
from jax.experimental import pallas as pl
from jax.experimental.pallas import tpu as pltpu
from jax.experimental.pallas import tpu_sc as plsc

VOCAB = 1000
D_EMB = 128
BATCH = 4096
HIST = 200

NC, NS = 2, 16
NW = NC * NS
ROWS = BATCH * HIST
RPW = ROWS // NW
CHUNK = 128
NCHUNK = RPW // CHUNK
NBUF = 4
LAG = 1


def _sc_body(table_hbm, idx_hbm, out_hbm, table_sp, idx_v,
             buf0, buf1, buf2, buf3, g0, g1, g2, g3, o0, o1, o2, o3):
    sid = lax.axis_index("s")
    wid = sid * NC + lax.axis_index("c")
    bufs = (buf0, buf1, buf2, buf3)
    gsems = (g0, g1, g2, g3)
    osems = (o0, o1, o2, o3)
    cbase = wid * NCHUNK

    @pl.when(sid == 0)
    def _():
        pltpu.sync_copy(table_hbm, table_sp)

    pltpu.sync_copy(idx_hbm.at[wid], idx_v)
    plsc.subcore_barrier()

    def g_start(g, b):
        pltpu.async_copy(table_sp.at[idx_v.at[g]], bufs[b], gsems[b])

    def g_wait(g, b):
        pltpu.make_async_copy(table_sp.at[idx_v.at[g]], bufs[b], gsems[b]).wait()

    def o_start(g, b):
        pltpu.async_copy(bufs[b], out_hbm.at[cbase + g], osems[b])

    def o_wait(g, b):
        pltpu.make_async_copy(bufs[b], out_hbm.at[cbase + g], osems[b]).wait()

    for g in range(LAG):
        g_start(g, g % NBUF)

    for g in range(NBUF):
        g_wait(g, g % NBUF)
        o_start(g, g % NBUF)
        if g + LAG - NBUF >= 0:
            o_wait(g + LAG - NBUF, (g + LAG) % NBUF)
        g_start(g + LAG, (g + LAG) % NBUF)

    def loop_body(i, carry):
        for b in range(NBUF):
            g = i * NBUF + b
            b2 = (b + LAG) % NBUF
            g_wait(g, b)
            o_start(g, b)
            o_wait(g + LAG - NBUF, b2)
            g_start(g + LAG, b2)
        return carry

    lax.fori_loop(1, NCHUNK // NBUF - 1, loop_body, None)

    for b in range(NBUF):
        g = NCHUNK - NBUF + b
        g_wait(g, b)
        o_start(g, b)
        if g + LAG < NCHUNK:
            o_wait(g + LAG - NBUF, (g + LAG) % NBUF)
            g_start(g + LAG, (g + LAG) % NBUF)

    for b in range(NBUF):
        g = NCHUNK - NBUF + b
        o_wait(g, b)


_sc_gather = functools.partial(
    pl.kernel,
    out_type=jax.ShapeDtypeStruct((NW * NCHUNK, CHUNK, D_EMB), jnp.float32),
    mesh=plsc.VectorSubcoreMesh(core_axis_name="c", subcore_axis_name="s"),
    scratch_types=(
        [pltpu.VMEM_SHARED((VOCAB, D_EMB), jnp.float32)]
        + [pltpu.VMEM((NCHUNK, CHUNK), jnp.int32)]
        + [pltpu.VMEM((CHUNK, D_EMB), jnp.float32)] * NBUF
        + [pltpu.SemaphoreType.DMA] * (2 * NBUF)
    ),
)(_sc_body)


def kernel(x, table):
    idx = x.astype(jnp.int32).reshape(NW, NCHUNK, CHUNK)
    out = _sc_gather(table, idx)
    return out.reshape(BATCH, HIST, D_EMB)


# speedup vs baseline: 1.3150x; 1.0676x over previous
import functools

import jax
import jax.numpy as jnp
from jax import lax
from jax.experimental import pallas as pl
from jax.experimental.pallas import tpu as pltpu
from jax.experimental.pallas import tpu_sc as plsc

VOCAB = 1000
D_EMB = 128
BATCH = 4096
HIST = 200

NC, NS = 2, 16
NW = NC * NS
ROWS = BATCH * HIST
RPW = ROWS // NW
CHUNK = 128
NCHUNK = RPW // CHUNK
NBUF = 5
LAG = 2


def _sc_body(table_hbm, idx_hbm, out_hbm, table_sp, idx_v,
             buf0, buf1, buf2, buf3, buf4,
             g0, g1, g2, g3, g4, o0, o1, o2, o3, o4):
    sid = lax.axis_index("s")
    wid = sid * NC + lax.axis_index("c")
    bufs = (buf0, buf1, buf2, buf3, buf4)
    gsems = (g0, g1, g2, g3, g4)
    osems = (o0, o1, o2, o3, o4)
    cbase = wid * NCHUNK

    @pl.when(sid == 0)
    def _():
        pltpu.sync_copy(table_hbm, table_sp)

    pltpu.sync_copy(idx_hbm.at[wid], idx_v)
    plsc.subcore_barrier()

    def g_start(g, b):
        pltpu.async_copy(table_sp.at[idx_v.at[g]], bufs[b], gsems[b])

    def g_wait(g, b):
        pltpu.make_async_copy(table_sp.at[idx_v.at[g]], bufs[b], gsems[b]).wait()

    def o_start(g, b):
        pltpu.async_copy(bufs[b], out_hbm.at[cbase + g], osems[b])

    def o_wait(g, b):
        pltpu.make_async_copy(bufs[b], out_hbm.at[cbase + g], osems[b]).wait()

    for g in range(LAG):
        g_start(g, g % NBUF)

    for g in range(NBUF):
        g_wait(g, g % NBUF)
        o_start(g, g % NBUF)
        if g + LAG - NBUF >= 0:
            o_wait(g + LAG - NBUF, (g + LAG) % NBUF)
        g_start(g + LAG, (g + LAG) % NBUF)

    def loop_body(i, carry):
        for b in range(NBUF):
            g = i * NBUF + b
            b2 = (b + LAG) % NBUF
            g_wait(g, b)
            o_start(g, b)
            o_wait(g + LAG - NBUF, b2)
            g_start(g + LAG, b2)
        return carry

    lax.fori_loop(1, NCHUNK // NBUF - 1, loop_body, None)

    for b in range(NBUF):
        g = NCHUNK - NBUF + b
        g_wait(g, b)
        o_start(g, b)
        if g + LAG < NCHUNK:
            o_wait(g + LAG - NBUF, (g + LAG) % NBUF)
            g_start(g + LAG, (g + LAG) % NBUF)

    for b in range(NBUF):
        g = NCHUNK - NBUF + b
        o_wait(g, b)


_sc_gather = functools.partial(
    pl.kernel,
    out_type=jax.ShapeDtypeStruct((NW * NCHUNK, CHUNK, D_EMB), jnp.float32),
    mesh=plsc.VectorSubcoreMesh(core_axis_name="c", subcore_axis_name="s"),
    scratch_types=(
        [pltpu.VMEM_SHARED((VOCAB, D_EMB), jnp.float32)]
        + [pltpu.VMEM((NCHUNK, CHUNK), jnp.int32)]
        + [pltpu.VMEM((CHUNK, D_EMB), jnp.float32)] * NBUF
        + [pltpu.SemaphoreType.DMA] * (2 * NBUF)
    ),
)(_sc_body)


def kernel(x, table):
    idx = x.astype(jnp.int32).reshape(NW, NCHUNK, CHUNK)
    out = _sc_gather(table, idx)
    return out.reshape(BATCH, HIST, D_EMB)


# speedup vs baseline: 1.3183x; 1.0025x over previous
import functools

import jax
import jax.numpy as jnp
from jax import lax
from jax.experimental import pallas as pl
from jax.experimental.pallas import tpu as pltpu
from jax.experimental.pallas import tpu_sc as plsc

VOCAB = 1000
D_EMB = 128
BATCH = 4096
HIST = 200

NC, NS = 2, 16
NW = NC * NS
ROWS = BATCH * HIST
RPW = ROWS // NW
CHUNK = 128
NCHUNK = RPW // CHUNK
NBUF = 5
LAG = 2


def _sc_body(table_hbm, idx_hbm, out_hbm, table_sp, idx_v,
             buf0, buf1, buf2, buf3, buf4,
             g0, g1, g2, g3, g4, o0, o1, o2, o3, o4):
    sid = lax.axis_index("s")
    wid = sid * NC + lax.axis_index("c")
    bufs = (buf0, buf1, buf2, buf3, buf4)
    gsems = (g0, g1, g2, g3, g4)
    osems = (o0, o1, o2, o3, o4)
    cbase = wid * NCHUNK

    @pl.when(sid < 10)
    def _():
        off = pl.multiple_of(sid * 96, 8)
        pltpu.sync_copy(table_hbm.at[pl.ds(off, 96)],
                        table_sp.at[pl.ds(off, 96)])

    @pl.when(sid == 10)
    def _():
        pltpu.sync_copy(table_hbm.at[pl.ds(960, 40)],
                        table_sp.at[pl.ds(960, 40)])

    pltpu.sync_copy(idx_hbm.at[wid], idx_v)
    plsc.subcore_barrier()

    def g_start(g, b):
        pltpu.async_copy(table_sp.at[idx_v.at[g]], bufs[b], gsems[b])

    def g_wait(g, b):
        pltpu.make_async_copy(table_sp.at[idx_v.at[g]], bufs[b], gsems[b]).wait()

    def o_start(g, b):
        pltpu.async_copy(bufs[b], out_hbm.at[cbase + g], osems[b])

    def o_wait(g, b):
        pltpu.make_async_copy(bufs[b], out_hbm.at[cbase + g], osems[b]).wait()

    for g in range(LAG):
        g_start(g, g % NBUF)

    for g in range(NBUF):
        g_wait(g, g % NBUF)
        o_start(g, g % NBUF)
        if g + LAG - NBUF >= 0:
            o_wait(g + LAG - NBUF, (g + LAG) % NBUF)
        g_start(g + LAG, (g + LAG) % NBUF)

    def loop_body(i, carry):
        for b in range(NBUF):
            g = i * NBUF + b
            b2 = (b + LAG) % NBUF
            g_wait(g, b)
            o_start(g, b)
            o_wait(g + LAG - NBUF, b2)
            g_start(g + LAG, b2)
        return carry

    lax.fori_loop(1, NCHUNK // NBUF - 1, loop_body, None)

    for b in range(NBUF):
        g = NCHUNK - NBUF + b
        g_wait(g, b)
        o_start(g, b)
        if g + LAG < NCHUNK:
            o_wait(g + LAG - NBUF, (g + LAG) % NBUF)
            g_start(g + LAG, (g + LAG) % NBUF)

    for b in range(NBUF):
        g = NCHUNK - NBUF + b
        o_wait(g, b)


_sc_gather = functools.partial(
    pl.kernel,
    out_type=jax.ShapeDtypeStruct((NW * NCHUNK, CHUNK, D_EMB), jnp.float32),
    mesh=plsc.VectorSubcoreMesh(core_axis_name="c", subcore_axis_name="s"),
    scratch_types=(
        [pltpu.VMEM_SHARED((VOCAB, D_EMB), jnp.float32)]
        + [pltpu.VMEM((NCHUNK, CHUNK), jnp.int32)]
        + [pltpu.VMEM((CHUNK, D_EMB), jnp.float32)] * NBUF
        + [pltpu.SemaphoreType.DMA] * (2 * NBUF)
    ),
)(_sc_body)


def kernel(x, table):
    idx = x.astype(jnp.int32).reshape(NW, NCHUNK, CHUNK)
    out = _sc_gather(table, idx)
    return out.reshape(BATCH, HIST, D_EMB)
